# baseline (speedup 1.0000x reference)
import functools

import jax
import jax.numpy as jnp
from jax import lax
from jax.experimental import pallas as pl
from jax.experimental.pallas import tpu as pltpu
from jax.experimental.pallas import tpu_sc as plsc

D_MODEL = 768
D_FF = 2048
NUM_EXPERTS = 16
TOKENS = 2048

BM = 256
MT = TOKENS // BM
NSTEP = MT + NUM_EXPERTS - 1
F32 = jnp.float32
BF16 = jnp.bfloat16


def _iota(shape, dim):
    return lax.broadcasted_iota(jnp.int32, shape, dim).astype(F32)


def _router_body(x_ref, r_ref, pos_ref, pf_ref):
    E = NUM_EXPERTS
    logits = jnp.dot(x_ref[...], r_ref[...], preferred_element_type=F32)
    mx = jnp.max(logits, axis=1, keepdims=True)
    col = _iota((TOKENS, E), 1)
    eid = jnp.min(jnp.where(logits == mx, col, float(E)), axis=1, keepdims=True)
    onehot = (col == eid).astype(F32)

    counts_row = jnp.dot(jnp.full((1, TOKENS), 1.0, F32), onehot,
                         preferred_element_type=F32)
    su = (_iota((E, E), 0) < _iota((E, E), 1)).astype(F32)
    starts_row = jnp.dot(counts_row, su, preferred_element_type=F32)
    ends_row = starts_row + counts_row

    G = 16
    GS = TOKENS // G
    oh3 = onehot.astype(BF16).reshape(G, GS, E)
    slb = (lax.broadcasted_iota(jnp.int32, (G, GS, GS), 2)
           < lax.broadcasted_iota(jnp.int32, (G, GS, GS), 1)).astype(BF16)
    local = lax.dot_general(slb, oh3, (((2,), (1,)), ((0,), (0,))),
                            preferred_element_type=F32).reshape(TOKENS, E)
    grp = (lax.broadcasted_iota(jnp.int32, (TOKENS, G), 0) // GS
           == lax.broadcasted_iota(jnp.int32, (TOKENS, G), 1))
    grp = grp.astype(F32)
    gsum = lax.dot_general(grp.astype(BF16), onehot.astype(BF16),
                           (((0,), (0,)), ((), ())),
                           preferred_element_type=F32)
    slg = (_iota((G, G), 1) < _iota((G, G), 0)).astype(F32)
    goff = jnp.dot(slg, gsum, preferred_element_type=F32)
    ranks = local + jnp.dot(grp, goff, preferred_element_type=F32)
    pos = jnp.sum((ranks + starts_row) * onehot, axis=1, keepdims=True)
    pos_ref[...] = pos.astype(jnp.int32).reshape(TOKENS)

    counts_col = lax.dot_general(onehot, jnp.full((TOKENS, 1), 1.0, F32),
                                 (((0,), (0,)), ((), ())),
                                 preferred_element_type=F32)
    slE = (_iota((E, E), 1) < _iota((E, E), 0)).astype(F32)
    starts_col = jnp.dot(slE, counts_col, preferred_element_type=F32)
    ends_col = starts_col + counts_col

    m_lane = _iota((E, MT), 1)
    inter_t = jnp.where((starts_col < (m_lane + 1.0) * BM)
                        & (ends_col > m_lane * BM), 1.0, 0.0)
    n_int_row = lax.dot_general(jnp.full((E, 1), 1.0, F32), inter_t,
                                (((0,), (0,)), ((), ())),
                                preferred_element_type=F32)
    u_incl = (_iota((MT, MT), 0) <= _iota((MT, MT), 1)).astype(F32)
    cum_m = jnp.dot(n_int_row, u_incl, preferred_element_type=F32)
    total = jnp.sum(n_int_row, axis=1, keepdims=True)

    t_row = _iota((NSTEP, 1), 0)
    tcl = jnp.minimum(t_row, total - 1.0)
    m_of = jnp.sum(jnp.where(cum_m <= tcl, 1.0, 0.0), 1, keepdims=True)
    m_idx = _iota((NSTEP, MT), 1)
    cum_before = jnp.sum(jnp.where(m_idx < m_of, 1.0, 0.0) * n_int_row,
                         1, keepdims=True)
    k_of = tcl - cum_before
    e_lane = _iota((NSTEP, E), 1)
    inter_s = jnp.where((starts_row < (m_of + 1.0) * BM)
                        & (ends_row > m_of * BM), 1.0, 0.0)
    uE = (_iota((E, E), 0) <= _iota((E, E), 1)).astype(F32)
    rowcum = jnp.dot(inter_s, uE, preferred_element_type=F32)
    e_of = jnp.sum(jnp.where(rowcum <= k_of, 1.0, 0.0), 1, keepdims=True)
    sel = jnp.where(e_lane == e_of, 1.0, 0.0)
    g_start = jnp.sum(sel * starts_row, 1, keepdims=True)
    g_end = jnp.sum(sel * ends_row, 1, keepdims=True)

    active = t_row < total
    neg = jnp.full((1, 1), -1.0, F32)
    m_prev = jnp.concatenate([neg, m_of[:-1]], axis=0)
    e_prev = jnp.concatenate([neg, e_of[:-1]], axis=0)
    first = jnp.where(active & (m_of != m_prev), 1.0, 0.0)
    new_e = jnp.where(active & (e_of != e_prev), 1.0, 0.0)
    g_start = jnp.where(active, g_start, 0.0)
    g_end = jnp.where(active, g_end, 0.0)

    eye = (_iota((NSTEP, NSTEP), 0) == _iota((NSTEP, NSTEP), 1)).astype(F32)
    e_row = lax.dot_general(e_of, eye, (((0,), (0,)), ((), ())),
                            preferred_element_type=F32)
    e_p1 = jnp.min(jnp.where(e_row > e_of, e_row, 99.0), 1, keepdims=True)
    e_p2 = jnp.min(jnp.where(e_row > e_p1, e_row, 99.0), 1, keepdims=True)
    e_p1 = jnp.where(e_p1 > 90.0, -1.0, e_p1)
    e_p2 = jnp.where(e_p2 > 90.0, -1.0, e_p2)
    tri = (_iota((NSTEP, NSTEP), 1) <= _iota((NSTEP, NSTEP), 0)).astype(F32)
    run_id = jnp.dot(tri, new_e, preferred_element_type=F32) - 1.0

    pf = jnp.concatenate([m_of, e_of, first, g_start, g_end, new_e,
                          e_p1, e_p2, run_id], axis=1)
    pf_ref[...] = pf.astype(jnp.int32)


def _router(flat, router):
    return pl.pallas_call(
        _router_body,
        out_shape=(
            jax.ShapeDtypeStruct((TOKENS,), jnp.int32),
            jax.ShapeDtypeStruct((NSTEP, 9), jnp.int32),
        ),
    )(flat, router)


def _gmm_body(pf_ref, x_ref, w1_hbm, w2_hbm, o_ref,
              w1s, w2s, w1b, w2b, s1, s2):
    t = pl.program_id(0)
    run = pf_ref[t, 8]
    slot = lax.rem(run, 3)

    def issue(e, s):
        pltpu.make_async_copy(w1_hbm.at[e], w1s.at[s], s1.at[s]).start()
        pltpu.make_async_copy(w2_hbm.at[e], w2s.at[s], s2.at[s]).start()

    @pl.when(t == 0)
    def _():
        issue(pf_ref[0, 1], 0)

        @pl.when(pf_ref[0, 6] >= 0)
        def _():
            issue(pf_ref[0, 6], 1)

        @pl.when(pf_ref[0, 7] >= 0)
        def _():
            issue(pf_ref[0, 7], 2)

    @pl.when((pf_ref[t, 5] == 1) & (t > 0) & (pf_ref[t, 7] >= 0))
    def _():
        issue(pf_ref[t, 7], lax.rem(run + 2, 3))

    @pl.when(pf_ref[t, 5] == 1)
    def _():
        e = pf_ref[t, 1]
        pltpu.make_async_copy(w1_hbm.at[e], w1s.at[slot], s1.at[slot]).wait()
        pltpu.make_async_copy(w2_hbm.at[e], w2s.at[slot], s2.at[slot]).wait()
        w1b[...] = w1s[slot].astype(BF16)
        w2b[...] = w2s[slot].astype(BF16)

    row0 = pf_ref[t, 0] * BM
    start = pf_ref[t, 3]
    end = pf_ref[t, 4]
    rows = row0 + lax.broadcasted_iota(jnp.int32, (BM, 1), 0)
    msk = (rows >= start) & (rows < end)
    x = jnp.where(msk, x_ref[...], 0.0).astype(BF16)
    h = jnp.dot(x, w1b[...], preferred_element_type=F32)
    h = (h * jax.nn.sigmoid(h)).astype(BF16)
    acc = jnp.dot(h, w2b[...], preferred_element_type=F32)

    @pl.when(pf_ref[t, 2] == 1)
    def _():
        o_ref[...] = acc

    @pl.when(pf_ref[t, 2] == 0)
    def _():
        o_ref[...] += acc


def _gmm(pf, x_sorted, expert_w1, expert_w2):
    grid_spec = pltpu.PrefetchScalarGridSpec(
        num_scalar_prefetch=1,
        grid=(NSTEP,),
        in_specs=[
            pl.BlockSpec((BM, D_MODEL), lambda t, pf: (pf[t, 0], 0)),
            pl.BlockSpec(memory_space=pl.ANY),
            pl.BlockSpec(memory_space=pl.ANY),
        ],
        out_specs=pl.BlockSpec((BM, D_MODEL), lambda t, pf: (pf[t, 0], 0)),
        scratch_shapes=[
            pltpu.VMEM((3, D_MODEL, D_FF), F32),
            pltpu.VMEM((3, D_FF, D_MODEL), F32),
            pltpu.VMEM((D_MODEL, D_FF), BF16),
            pltpu.VMEM((D_FF, D_MODEL), BF16),
            pltpu.SemaphoreType.DMA((3,)),
            pltpu.SemaphoreType.DMA((3,)),
        ],
    )
    return pl.pallas_call(
        _gmm_body,
        grid_spec=grid_spec,
        out_shape=jax.ShapeDtypeStruct((TOKENS, D_MODEL), jnp.float32),
    )(pf, x_sorted, expert_w1, expert_w2)


def _sc_mesh_info():
    info = plsc.get_sparse_core_info()
    nw = info.num_cores * info.num_subcores
    mesh = plsc.VectorSubcoreMesh(core_axis_name="c", subcore_axis_name="s")
    return info, nw, mesh


_SC_CH = 4


def _sc_scatter(table, idx):
    info, nw, mesh = _sc_mesh_info()
    b, d = idx.shape[0], table.shape[1]
    b_per_w = b // nw
    ch = b_per_w // _SC_CH

    @functools.partial(
        pl.kernel,
        mesh=mesh,
        out_type=jax.ShapeDtypeStruct((b, d), jnp.float32),
        scratch_types=[
            pltpu.VMEM((_SC_CH, ch), jnp.int32),
            pltpu.VMEM((b_per_w, d), jnp.float32),
            pltpu.SemaphoreType.DMA((_SC_CH,)),
            pltpu.SemaphoreType.DMA((_SC_CH,)),
        ],
    )
    def k(table_hbm, idx_hbm, out_hbm, idx_v, rows_v, rsem, wsem):
        wid = lax.axis_index("s") * info.num_cores + lax.axis_index("c")
        base = wid * b_per_w
        for c in range(_SC_CH):
            pltpu.sync_copy(idx_hbm.at[pl.ds(base + c * ch, ch)], idx_v.at[c])
        for c in range(_SC_CH):
            pltpu.make_async_copy(table_hbm.at[pl.ds(base + c * ch, ch)],
                                  rows_v.at[pl.ds(c * ch, ch)],
                                  rsem.at[c]).start()
        for c in range(_SC_CH):
            pltpu.make_async_copy(table_hbm.at[pl.ds(base + c * ch, ch)],
                                  rows_v.at[pl.ds(c * ch, ch)],
                                  rsem.at[c]).wait()
            pltpu.make_async_copy(rows_v.at[pl.ds(c * ch, ch)],
                                  out_hbm.at[idx_v.at[c]],
                                  wsem.at[c]).start()
        for c in range(_SC_CH):
            pltpu.make_async_copy(rows_v.at[pl.ds(c * ch, ch)],
                                  out_hbm.at[idx_v.at[c]],
                                  wsem.at[c]).wait()

    return k(table, idx)


def _sc_gather(table, idx):
    info, nw, mesh = _sc_mesh_info()
    b, d = idx.shape[0], table.shape[1]
    b_per_w = b // nw
    ch = b_per_w // _SC_CH

    @functools.partial(
        pl.kernel,
        mesh=mesh,
        out_type=jax.ShapeDtypeStruct((b, d), jnp.float32),
        scratch_types=[
            pltpu.VMEM((_SC_CH, ch), jnp.int32),
            pltpu.VMEM((b_per_w, d), jnp.float32),
            pltpu.SemaphoreType.DMA((_SC_CH,)),
            pltpu.SemaphoreType.DMA((_SC_CH,)),
        ],
    )
    def k(table_hbm, idx_hbm, out_hbm, idx_v, rows_v, gsem, ssem):
        wid = lax.axis_index("s") * info.num_cores + lax.axis_index("c")
        base = wid * b_per_w
        for c in range(_SC_CH):
            pltpu.sync_copy(idx_hbm.at[pl.ds(base + c * ch, ch)], idx_v.at[c])
        for c in range(_SC_CH):
            pltpu.make_async_copy(table_hbm.at[idx_v.at[c]],
                                  rows_v.at[pl.ds(c * ch, ch)],
                                  gsem.at[c]).start()
        for c in range(_SC_CH):
            pltpu.make_async_copy(table_hbm.at[idx_v.at[c]],
                                  rows_v.at[pl.ds(c * ch, ch)],
                                  gsem.at[c]).wait()
            pltpu.make_async_copy(rows_v.at[pl.ds(c * ch, ch)],
                                  out_hbm.at[pl.ds(base + c * ch, ch)],
                                  ssem.at[c]).start()
        for c in range(_SC_CH):
            pltpu.make_async_copy(rows_v.at[pl.ds(c * ch, ch)],
                                  out_hbm.at[pl.ds(base + c * ch, ch)],
                                  ssem.at[c]).wait()

    return k(table, idx)


def kernel(inputs, router, expert_w1, expert_w2):
    flat = inputs.reshape(TOKENS, D_MODEL)
    pos, pf = _router(flat, router)
    x_sorted = _sc_scatter(flat, pos)
    out_sorted = _gmm(pf, x_sorted, expert_w1, expert_w2)
    out = _sc_gather(out_sorted, pos)
    return out.reshape(inputs.shape)


# speedup vs baseline: 1.0267x; 1.0267x over previous
import functools

import jax
import jax.numpy as jnp
from jax import lax
from jax.experimental import pallas as pl
from jax.experimental.pallas import tpu as pltpu
from jax.experimental.pallas import tpu_sc as plsc

D_MODEL = 768
D_FF = 2048
NUM_EXPERTS = 16
TOKENS = 2048

BM = 256
MT = TOKENS // BM
NSTEP = MT + NUM_EXPERTS - 1
F32 = jnp.float32
BF16 = jnp.bfloat16


def _iota(shape, dim):
    return lax.broadcasted_iota(jnp.int32, shape, dim).astype(F32)


def _router_body(x_ref, r_ref, pos_ref, pf_ref):
    E = NUM_EXPERTS
    logits = jnp.dot(x_ref[...], r_ref[...], preferred_element_type=F32)
    mx = jnp.max(logits, axis=1, keepdims=True)
    col = _iota((TOKENS, E), 1)
    eid = jnp.min(jnp.where(logits == mx, col, float(E)), axis=1, keepdims=True)
    onehot = (col == eid).astype(F32)

    counts_row = jnp.dot(jnp.full((1, TOKENS), 1.0, F32), onehot,
                         preferred_element_type=F32)
    su = (_iota((E, E), 0) < _iota((E, E), 1)).astype(F32)
    starts_row = jnp.dot(counts_row, su, preferred_element_type=F32)
    ends_row = starts_row + counts_row

    G = 16
    GS = TOKENS // G
    oh3 = onehot.astype(BF16).reshape(G, GS, E)
    slb = (lax.broadcasted_iota(jnp.int32, (G, GS, GS), 2)
           < lax.broadcasted_iota(jnp.int32, (G, GS, GS), 1)).astype(BF16)
    local = lax.dot_general(slb, oh3, (((2,), (1,)), ((0,), (0,))),
                            preferred_element_type=F32).reshape(TOKENS, E)
    grp = (lax.broadcasted_iota(jnp.int32, (TOKENS, G), 0) // GS
           == lax.broadcasted_iota(jnp.int32, (TOKENS, G), 1))
    grp = grp.astype(F32)
    gsum = lax.dot_general(grp.astype(BF16), onehot.astype(BF16),
                           (((0,), (0,)), ((), ())),
                           preferred_element_type=F32)
    slg = (_iota((G, G), 1) < _iota((G, G), 0)).astype(F32)
    goff = jnp.dot(slg, gsum, preferred_element_type=F32)
    ranks = local + jnp.dot(grp, goff, preferred_element_type=F32)
    pos = jnp.sum((ranks + starts_row) * onehot, axis=1, keepdims=True)
    pos_ref[...] = pos.astype(jnp.int32).reshape(TOKENS)

    counts_col = lax.dot_general(onehot, jnp.full((TOKENS, 1), 1.0, F32),
                                 (((0,), (0,)), ((), ())),
                                 preferred_element_type=F32)
    slE = (_iota((E, E), 1) < _iota((E, E), 0)).astype(F32)
    starts_col = jnp.dot(slE, counts_col, preferred_element_type=F32)
    ends_col = starts_col + counts_col

    m_lane = _iota((E, MT), 1)
    inter_t = jnp.where((starts_col < (m_lane + 1.0) * BM)
                        & (ends_col > m_lane * BM), 1.0, 0.0)
    n_int_row = lax.dot_general(jnp.full((E, 1), 1.0, F32), inter_t,
                                (((0,), (0,)), ((), ())),
                                preferred_element_type=F32)
    u_incl = (_iota((MT, MT), 0) <= _iota((MT, MT), 1)).astype(F32)
    cum_m = jnp.dot(n_int_row, u_incl, preferred_element_type=F32)
    total = jnp.sum(n_int_row, axis=1, keepdims=True)

    t_row = _iota((NSTEP, 1), 0)
    tcl = jnp.minimum(t_row, total - 1.0)
    m_of = jnp.sum(jnp.where(cum_m <= tcl, 1.0, 0.0), 1, keepdims=True)
    m_idx = _iota((NSTEP, MT), 1)
    cum_before = jnp.sum(jnp.where(m_idx < m_of, 1.0, 0.0) * n_int_row,
                         1, keepdims=True)
    k_of = tcl - cum_before
    e_lane = _iota((NSTEP, E), 1)
    inter_s = jnp.where((starts_row < (m_of + 1.0) * BM)
                        & (ends_row > m_of * BM), 1.0, 0.0)
    uE = (_iota((E, E), 0) <= _iota((E, E), 1)).astype(F32)
    rowcum = jnp.dot(inter_s, uE, preferred_element_type=F32)
    e_of = jnp.sum(jnp.where(rowcum <= k_of, 1.0, 0.0), 1, keepdims=True)
    sel = jnp.where(e_lane == e_of, 1.0, 0.0)
    g_start = jnp.sum(sel * starts_row, 1, keepdims=True)
    g_end = jnp.sum(sel * ends_row, 1, keepdims=True)

    active = t_row < total
    neg = jnp.full((1, 1), -1.0, F32)
    m_prev = jnp.concatenate([neg, m_of[:-1]], axis=0)
    e_prev = jnp.concatenate([neg, e_of[:-1]], axis=0)
    first = jnp.where(active & (m_of != m_prev), 1.0, 0.0)
    new_e = jnp.where(active & (e_of != e_prev), 1.0, 0.0)
    g_start = jnp.where(active, g_start, 0.0)
    g_end = jnp.where(active, g_end, 0.0)

    eye = (_iota((NSTEP, NSTEP), 0) == _iota((NSTEP, NSTEP), 1)).astype(F32)
    e_row = lax.dot_general(e_of, eye, (((0,), (0,)), ((), ())),
                            preferred_element_type=F32)
    e_p1 = jnp.min(jnp.where(e_row > e_of, e_row, 99.0), 1, keepdims=True)
    e_p2 = jnp.min(jnp.where(e_row > e_p1, e_row, 99.0), 1, keepdims=True)
    e_p1 = jnp.where(e_p1 > 90.0, -1.0, e_p1)
    e_p2 = jnp.where(e_p2 > 90.0, -1.0, e_p2)
    tri = (_iota((NSTEP, NSTEP), 1) <= _iota((NSTEP, NSTEP), 0)).astype(F32)
    run_id = jnp.dot(tri, new_e, preferred_element_type=F32) - 1.0

    pf = jnp.concatenate([m_of, e_of, first, g_start, g_end, new_e,
                          e_p1, e_p2, run_id], axis=1)
    pf_ref[...] = pf.astype(jnp.int32)


def _router(flat, router):
    return pl.pallas_call(
        _router_body,
        out_shape=(
            jax.ShapeDtypeStruct((TOKENS,), jnp.int32),
            jax.ShapeDtypeStruct((NSTEP, 9), jnp.int32),
        ),
    )(flat, router)


def _gmm_body(pf_ref, x_ref, w1_hbm, w2_hbm, o_ref,
              w1s, w2s, w1b, w2b, s1, s2):
    t = pl.program_id(0)
    run = pf_ref[t, 8]
    slot = lax.rem(run, 3)

    def issue(e, s):
        pltpu.make_async_copy(w1_hbm.at[e], w1s.at[s], s1.at[s]).start()
        pltpu.make_async_copy(w2_hbm.at[e], w2s.at[s], s2.at[s]).start()

    @pl.when(t == 0)
    def _():
        issue(pf_ref[0, 1], 0)

        @pl.when(pf_ref[0, 6] >= 0)
        def _():
            issue(pf_ref[0, 6], 1)

        @pl.when(pf_ref[0, 7] >= 0)
        def _():
            issue(pf_ref[0, 7], 2)

    @pl.when((pf_ref[t, 5] == 1) & (t > 0) & (pf_ref[t, 7] >= 0))
    def _():
        issue(pf_ref[t, 7], lax.rem(run + 2, 3))

    @pl.when(pf_ref[t, 5] == 1)
    def _():
        e = pf_ref[t, 1]
        pltpu.make_async_copy(w1_hbm.at[e], w1s.at[slot], s1.at[slot]).wait()
        pltpu.make_async_copy(w2_hbm.at[e], w2s.at[slot], s2.at[slot]).wait()
        w1b[...] = w1s[slot].astype(BF16)
        w2b[...] = w2s[slot].astype(BF16)

    row0 = pf_ref[t, 0] * BM
    start = pf_ref[t, 3]
    end = pf_ref[t, 4]
    rows = row0 + lax.broadcasted_iota(jnp.int32, (BM, 1), 0)
    msk = (rows >= start) & (rows < end)
    x = jnp.where(msk, x_ref[...], 0.0).astype(BF16)
    h = jnp.dot(x, w1b[...], preferred_element_type=F32)
    h = (h * jax.nn.sigmoid(h)).astype(BF16)
    acc = jnp.dot(h, w2b[...], preferred_element_type=F32)

    @pl.when(pf_ref[t, 2] == 1)
    def _():
        o_ref[...] = acc

    @pl.when(pf_ref[t, 2] == 0)
    def _():
        o_ref[...] += acc


def _gmm(pf, x_sorted, expert_w1, expert_w2):
    grid_spec = pltpu.PrefetchScalarGridSpec(
        num_scalar_prefetch=1,
        grid=(NSTEP,),
        in_specs=[
            pl.BlockSpec((BM, D_MODEL), lambda t, pf: (pf[t, 0], 0)),
            pl.BlockSpec(memory_space=pl.ANY),
            pl.BlockSpec(memory_space=pl.ANY),
        ],
        out_specs=pl.BlockSpec((BM, D_MODEL), lambda t, pf: (pf[t, 0], 0)),
        scratch_shapes=[
            pltpu.VMEM((3, D_MODEL, D_FF), F32),
            pltpu.VMEM((3, D_FF, D_MODEL), F32),
            pltpu.VMEM((D_MODEL, D_FF), BF16),
            pltpu.VMEM((D_FF, D_MODEL), BF16),
            pltpu.SemaphoreType.DMA((3,)),
            pltpu.SemaphoreType.DMA((3,)),
        ],
    )
    return pl.pallas_call(
        _gmm_body,
        grid_spec=grid_spec,
        out_shape=jax.ShapeDtypeStruct((TOKENS, D_MODEL), jnp.float32),
    )(pf, x_sorted, expert_w1, expert_w2)


def _sc_mesh_info():
    info = plsc.get_sparse_core_info()
    nw = info.num_cores * info.num_subcores
    mesh = plsc.VectorSubcoreMesh(core_axis_name="c", subcore_axis_name="s")
    return info, nw, mesh


def _sc_scatter(table, idx):
    info, nw, mesh = _sc_mesh_info()
    b, d = idx.shape[0], table.shape[1]
    b_per_w = b // nw

    @functools.partial(
        pl.kernel,
        mesh=mesh,
        out_type=jax.ShapeDtypeStruct((b, d), jnp.float32),
        scratch_types=[
            pltpu.VMEM((b_per_w,), jnp.int32),
            pltpu.VMEM((b_per_w, d), jnp.float32),
            pltpu.SemaphoreType.DMA,
        ],
    )
    def k(table_hbm, idx_hbm, out_hbm, idx_v, rows_v, sem):
        wid = lax.axis_index("s") * info.num_cores + lax.axis_index("c")
        base = wid * b_per_w
        pltpu.sync_copy(idx_hbm.at[pl.ds(base, b_per_w)], idx_v)
        pltpu.sync_copy(table_hbm.at[pl.ds(base, b_per_w)], rows_v)
        pltpu.async_copy(rows_v, out_hbm.at[idx_v], sem).wait()

    return k(table, idx)


def _sc_gather(table, idx):
    info, nw, mesh = _sc_mesh_info()
    b, d = idx.shape[0], table.shape[1]
    b_per_w = b // nw

    @functools.partial(
        pl.kernel,
        mesh=mesh,
        out_type=jax.ShapeDtypeStruct((b, d), jnp.float32),
        scratch_types=[
            pltpu.VMEM((b_per_w,), jnp.int32),
            pltpu.VMEM((b_per_w, d), jnp.float32),
            pltpu.SemaphoreType.DMA,
        ],
    )
    def k(table_hbm, idx_hbm, out_hbm, idx_v, rows_v, sem):
        wid = lax.axis_index("s") * info.num_cores + lax.axis_index("c")
        base = wid * b_per_w
        pltpu.sync_copy(idx_hbm.at[pl.ds(base, b_per_w)], idx_v)
        pltpu.async_copy(table_hbm.at[idx_v], rows_v, sem).wait()
        pltpu.sync_copy(rows_v, out_hbm.at[pl.ds(base, b_per_w)])

    return k(table, idx)


def kernel(inputs, router, expert_w1, expert_w2):
    flat = inputs.reshape(TOKENS, D_MODEL)
    pos, pf = _router(flat, router)
    x_sorted = _sc_scatter(flat, pos)
    out_sorted = _gmm(pf, x_sorted, expert_w1, expert_w2)
    out = _sc_gather(out_sorted, pos)
    return out.reshape(inputs.shape)
